# baseline (device time: 931714 ns/iter reference)
import jax
import jax.numpy as jnp
from jax import lax
from jax.experimental import pallas as pl
from jax.experimental.pallas import tpu as pltpu

N_DEV = 32
E_LOC = 4
N_EXP = 128
D_MODEL = 256
D_HID = 512
N_TOK = 1024


def kernel(x, router_W, route_idx, expert_W):
    assert x.shape == (N_TOK, D_MODEL), x.shape
    assert expert_W.shape == (E_LOC, D_MODEL, D_HID), expert_W.shape

    def body(x_ref, rw_ref, idx_ref, ew_ref, out_ref,
             comm_ref, send_sems, recv_sems, credit_sem):
        my = lax.axis_index("i")
        left = lax.rem(my - 1 + N_DEV, N_DEV)
        right = lax.rem(my + 1, N_DEV)

        bsem = pltpu.get_barrier_semaphore()
        for nbr in (left, right):
            pl.semaphore_signal(bsem, inc=1, device_id=(nbr,),
                                device_id_type=pl.DeviceIdType.MESH)
        pl.semaphore_wait(bsem, 2)

        xv = x_ref[...]
        scores = jnp.dot(xv, rw_ref[...], preferred_element_type=jnp.float32)
        smax = jnp.max(scores, axis=1, keepdims=True)
        p = jnp.exp(scores - smax)
        p = p / jnp.sum(p, axis=1, keepdims=True)
        idx0 = idx_ref[:, 0:1]
        idx1 = idx_ref[:, 1:2]
        eids = lax.broadcasted_iota(jnp.int32, (N_TOK, N_EXP), 1)
        g0 = jnp.sum(jnp.where(eids == idx0, p, 0.0), axis=1, keepdims=True)
        g1 = jnp.sum(jnp.where(eids == idx1, p, 0.0), axis=1, keepdims=True)
        gs = g0 + g1
        w0 = g0 / gs
        w1 = g1 / gs

        comm_ref[0] = ew_ref[...].reshape(E_LOC * D_MODEL, D_HID)

        acc = jnp.zeros((N_TOK, D_HID), jnp.float32)
        for h in range(N_DEV):
            c = h % 2
            r = (h + 1) % 2
            rdma = None
            if h < N_DEV - 1:
                if h >= 1:
                    pl.semaphore_wait(credit_sem, 1)
                rdma = pltpu.make_async_remote_copy(
                    src_ref=comm_ref.at[c],
                    dst_ref=comm_ref.at[r],
                    send_sem=send_sems.at[c],
                    recv_sem=recv_sems.at[r],
                    device_id=(right,),
                    device_id_type=pl.DeviceIdType.MESH,
                )
                rdma.start()

            origin = lax.rem(my - h + N_DEV, N_DEV)
            parts = []
            for e in range(E_LOC):
                gid = origin * E_LOC + e
                m = (jnp.where(idx0 == gid, w0, 0.0)
                     + jnp.where(idx1 == gid, w1, 0.0))
                parts.append(xv * m)
            xw = jnp.concatenate(parts, axis=1)
            acc = acc + jnp.dot(xw, comm_ref[c],
                                preferred_element_type=jnp.float32)

            if h < N_DEV - 1:
                rdma.wait()
                if h < N_DEV - 2:
                    pl.semaphore_signal(credit_sem, inc=1, device_id=(left,),
                                        device_id_type=pl.DeviceIdType.MESH)

        out_ref[...] = acc

    return pl.pallas_call(
        body,
        out_shape=jax.ShapeDtypeStruct((N_TOK, D_HID), jnp.float32),
        in_specs=[
            pl.BlockSpec(memory_space=pltpu.VMEM),
            pl.BlockSpec(memory_space=pltpu.VMEM),
            pl.BlockSpec(memory_space=pltpu.VMEM),
            pl.BlockSpec(memory_space=pltpu.VMEM),
        ],
        out_specs=pl.BlockSpec(memory_space=pltpu.VMEM),
        scratch_shapes=[
            pltpu.VMEM((2, E_LOC * D_MODEL, D_HID), jnp.float32),
            pltpu.SemaphoreType.DMA((2,)),
            pltpu.SemaphoreType.DMA((2,)),
            pltpu.SemaphoreType.REGULAR,
        ],
        compiler_params=pltpu.CompilerParams(collective_id=0),
    )(x, router_W, route_idx, expert_W)


# device time: 580424 ns/iter; 1.6052x vs baseline; 1.6052x over previous
import jax
import jax.numpy as jnp
from jax import lax
from jax.experimental import pallas as pl
from jax.experimental.pallas import tpu as pltpu

N_DEV = 32
E_LOC = 4
N_EXP = 128
D_MODEL = 256
D_HID = 512
N_TOK = 1024


def kernel(x, router_W, route_idx, expert_W):
    assert x.shape == (N_TOK, D_MODEL), x.shape
    assert expert_W.shape == (E_LOC, D_MODEL, D_HID), expert_W.shape

    def body(x_ref, rw_ref, idx_ref, ew_ref, out_ref,
             comm_ref, send_sems, recv_sems, credit_sem):
        my = lax.axis_index("i")
        left = lax.rem(my - 1 + N_DEV, N_DEV)
        right = lax.rem(my + 1, N_DEV)

        bsem = pltpu.get_barrier_semaphore()
        for nbr in (left, right):
            pl.semaphore_signal(bsem, inc=1, device_id=(nbr,),
                                device_id_type=pl.DeviceIdType.MESH)
        pl.semaphore_wait(bsem, 2)

        xv = x_ref[...]
        scores = jnp.dot(xv, rw_ref[...], preferred_element_type=jnp.float32)
        smax = jnp.max(scores, axis=1, keepdims=True)
        p = jnp.exp(scores - smax)
        p = p / jnp.sum(p, axis=1, keepdims=True)
        idx0 = idx_ref[:, 0:1]
        idx1 = idx_ref[:, 1:2]
        eids = lax.broadcasted_iota(jnp.int32, (N_TOK, N_EXP), 1)
        g0 = jnp.sum(jnp.where(eids == idx0, p, 0.0), axis=1, keepdims=True)
        g1 = jnp.sum(jnp.where(eids == idx1, p, 0.0), axis=1, keepdims=True)
        gs = g0 + g1
        w0 = g0 / gs
        w1 = g1 / gs

        comm_ref[0] = ew_ref[...].reshape(E_LOC * D_MODEL, D_HID).astype(jnp.bfloat16)

        acc = jnp.zeros((N_TOK, D_HID), jnp.float32)
        for h in range(N_DEV):
            c = h % 2
            r = (h + 1) % 2
            rdma = None
            if h < N_DEV - 1:
                if h >= 1:
                    pl.semaphore_wait(credit_sem, 1)
                rdma = pltpu.make_async_remote_copy(
                    src_ref=comm_ref.at[c],
                    dst_ref=comm_ref.at[r],
                    send_sem=send_sems.at[c],
                    recv_sem=recv_sems.at[r],
                    device_id=(right,),
                    device_id_type=pl.DeviceIdType.MESH,
                )
                rdma.start()

            origin = lax.rem(my - h + N_DEV, N_DEV)
            parts = []
            for e in range(E_LOC):
                gid = origin * E_LOC + e
                m = (jnp.where(idx0 == gid, w0, 0.0)
                     + jnp.where(idx1 == gid, w1, 0.0))
                parts.append((xv * m).astype(jnp.bfloat16))
            xw = jnp.concatenate(parts, axis=1)
            acc = acc + jnp.dot(xw, comm_ref[c],
                                preferred_element_type=jnp.float32)

            if h < N_DEV - 1:
                rdma.wait()
                if h < N_DEV - 2:
                    pl.semaphore_signal(credit_sem, inc=1, device_id=(left,),
                                        device_id_type=pl.DeviceIdType.MESH)

        out_ref[...] = acc

    return pl.pallas_call(
        body,
        out_shape=jax.ShapeDtypeStruct((N_TOK, D_HID), jnp.float32),
        in_specs=[
            pl.BlockSpec(memory_space=pltpu.VMEM),
            pl.BlockSpec(memory_space=pltpu.VMEM),
            pl.BlockSpec(memory_space=pltpu.VMEM),
            pl.BlockSpec(memory_space=pltpu.VMEM),
        ],
        out_specs=pl.BlockSpec(memory_space=pltpu.VMEM),
        scratch_shapes=[
            pltpu.VMEM((2, E_LOC * D_MODEL, D_HID), jnp.bfloat16),
            pltpu.SemaphoreType.DMA((2,)),
            pltpu.SemaphoreType.DMA((2,)),
            pltpu.SemaphoreType.REGULAR,
        ],
        compiler_params=pltpu.CompilerParams(collective_id=0),
    )(x, router_W, route_idx, expert_W)


# device time: 411946 ns/iter; 2.2617x vs baseline; 1.4090x over previous
import jax
import jax.numpy as jnp
from jax import lax
from jax.experimental import pallas as pl
from jax.experimental.pallas import tpu as pltpu

N_DEV = 32
E_LOC = 4
N_EXP = 128
D_MODEL = 256
D_HID = 512
N_TOK = 1024
DEPTH = 4


def kernel(x, router_W, route_idx, expert_W):
    assert x.shape == (N_TOK, D_MODEL), x.shape
    assert expert_W.shape == (E_LOC, D_MODEL, D_HID), expert_W.shape

    def body(x_ref, rw_ref, idx_ref, ew_ref, out_ref,
             comm_r, comm_l, send_r, recv_r, send_l, recv_l,
             cred_r, cred_l):
        my = lax.axis_index("i")
        left = lax.rem(my - 1 + N_DEV, N_DEV)
        right = lax.rem(my + 1, N_DEV)

        bsem = pltpu.get_barrier_semaphore()
        for nbr in (left, right):
            pl.semaphore_signal(bsem, inc=1, device_id=(nbr,),
                                device_id_type=pl.DeviceIdType.MESH)
        pl.semaphore_wait(bsem, 2)

        xv = x_ref[...]
        scores = jnp.dot(xv, rw_ref[...], preferred_element_type=jnp.float32)
        smax = jnp.max(scores, axis=1, keepdims=True)
        p = jnp.exp(scores - smax)
        p = p / jnp.sum(p, axis=1, keepdims=True)
        idx0 = idx_ref[:, 0:1]
        idx1 = idx_ref[:, 1:2]
        eids = lax.broadcasted_iota(jnp.int32, (N_TOK, N_EXP), 1)
        g0 = jnp.sum(jnp.where(eids == idx0, p, 0.0), axis=1, keepdims=True)
        g1 = jnp.sum(jnp.where(eids == idx1, p, 0.0), axis=1, keepdims=True)
        gs = g0 + g1
        w0 = g0 / gs
        w1 = g1 / gs

        ew = ew_ref[...]
        comm_r[0] = ew[0:2].reshape(2 * D_MODEL, D_HID).astype(jnp.bfloat16)
        comm_l[0] = ew[2:4].reshape(2 * D_MODEL, D_HID).astype(jnp.bfloat16)

        def masked_x(origin, experts):
            parts = []
            for e in experts:
                gid = origin * E_LOC + e
                m = (jnp.where(idx0 == gid, w0, 0.0)
                     + jnp.where(idx1 == gid, w1, 0.0))
                parts.append((xv * m).astype(jnp.bfloat16))
            return jnp.concatenate(parts, axis=1)

        acc = jnp.zeros((N_TOK, D_HID), jnp.float32)
        for h in range(N_DEV):
            c = h % DEPTH
            r = (h + 1) % DEPTH
            rdma_right = rdma_left = None
            if h < N_DEV - 1:
                if h >= DEPTH - 1:
                    pl.semaphore_wait(cred_r, 1)
                    pl.semaphore_wait(cred_l, 1)
                rdma_right = pltpu.make_async_remote_copy(
                    src_ref=comm_r.at[c], dst_ref=comm_r.at[r],
                    send_sem=send_r.at[c], recv_sem=recv_r.at[r],
                    device_id=(right,),
                    device_id_type=pl.DeviceIdType.MESH,
                )
                rdma_right.start()
                rdma_left = pltpu.make_async_remote_copy(
                    src_ref=comm_l.at[c], dst_ref=comm_l.at[r],
                    send_sem=send_l.at[c], recv_sem=recv_l.at[r],
                    device_id=(left,),
                    device_id_type=pl.DeviceIdType.MESH,
                )
                rdma_left.start()

            origin_r = lax.rem(my - h + N_DEV, N_DEV)
            origin_l = lax.rem(my + h, N_DEV)
            acc = acc + jnp.dot(masked_x(origin_r, (0, 1)), comm_r[c],
                                preferred_element_type=jnp.float32)
            acc = acc + jnp.dot(masked_x(origin_l, (2, 3)), comm_l[c],
                                preferred_element_type=jnp.float32)

            if h < N_DEV - 1:
                rdma_right.wait()
                rdma_left.wait()
                if h < N_DEV - DEPTH:
                    pl.semaphore_signal(cred_r, inc=1, device_id=(left,),
                                        device_id_type=pl.DeviceIdType.MESH)
                    pl.semaphore_signal(cred_l, inc=1, device_id=(right,),
                                        device_id_type=pl.DeviceIdType.MESH)

        out_ref[...] = acc

    return pl.pallas_call(
        body,
        out_shape=jax.ShapeDtypeStruct((N_TOK, D_HID), jnp.float32),
        in_specs=[
            pl.BlockSpec(memory_space=pltpu.VMEM),
            pl.BlockSpec(memory_space=pltpu.VMEM),
            pl.BlockSpec(memory_space=pltpu.VMEM),
            pl.BlockSpec(memory_space=pltpu.VMEM),
        ],
        out_specs=pl.BlockSpec(memory_space=pltpu.VMEM),
        scratch_shapes=[
            pltpu.VMEM((DEPTH, 2 * D_MODEL, D_HID), jnp.bfloat16),
            pltpu.VMEM((DEPTH, 2 * D_MODEL, D_HID), jnp.bfloat16),
            pltpu.SemaphoreType.DMA((DEPTH,)),
            pltpu.SemaphoreType.DMA((DEPTH,)),
            pltpu.SemaphoreType.DMA((DEPTH,)),
            pltpu.SemaphoreType.DMA((DEPTH,)),
            pltpu.SemaphoreType.REGULAR,
            pltpu.SemaphoreType.REGULAR,
        ],
        compiler_params=pltpu.CompilerParams(collective_id=0),
    )(x, router_W, route_idx, expert_W)
